# baseline (speedup 1.0000x reference)
import jax
import jax.numpy as jnp
from jax.experimental import pallas as pl
from jax.experimental.pallas import tpu as pltpu

T = 2048
H = 1024
E = 8
I = 512
ISH = 1024
SCH = ISH // E
ROUTED_SCALING = 2.5

NQ = 2
Q = T // NQ


def _dot_t(a, b):
    return jax.lax.dot_general(
        a, b, (((1,), (1,)), ((), ())), preferred_element_type=jnp.float32)


def _argmax8_lowest(vals):
    best = vals[:, 0:1]
    bidx = jnp.zeros_like(best, dtype=jnp.int32)
    for c in range(1, E):
        v = vals[:, c : c + 1]
        take = v > best
        bidx = jnp.where(take, jnp.int32(c), bidx)
        best = jnp.where(take, v, best)
    return bidx


def _routing_weights(xb, gate_w, e_bias, q):
    logits = _dot_t(xb, gate_w.astype(jnp.bfloat16))
    scores = jax.nn.sigmoid(logits)
    s4c = scores + e_bias

    def top2sum(g):
        cols = [g[:, c : c + 1] for c in range(4)]
        ps = [cols[i] + cols[j] for i in range(4) for j in range(i + 1, 4)]
        out = ps[0]
        for p in ps[1:]:
            out = jnp.maximum(out, p)
        return out

    one = jnp.float32(1.0)
    zero = jnp.float32(0.0)
    use_g0 = jnp.where(top2sum(s4c[:, 0:4]) >= top2sum(s4c[:, 4:8]), one, zero)
    col = jax.lax.broadcasted_iota(jnp.int32, (q, E), 1)
    in_g0 = jnp.where(col < 4, one, zero)
    group_mask = in_g0 * use_g0 + (one - in_g0) * (one - use_g0)
    neg_inf = jnp.float32(-jnp.inf)
    masked = jnp.where(group_mask > 0.5, s4c, neg_inf)

    idx1 = _argmax8_lowest(masked)
    idx2 = _argmax8_lowest(jnp.where(col == idx1, neg_inf, masked))
    oh1 = jnp.where(col == idx1, one, zero)
    oh2 = jnp.where(col == idx2, one, zero)
    w1 = jnp.sum(oh1 * scores, axis=1, keepdims=True)
    w2 = jnp.sum(oh2 * scores, axis=1, keepdims=True)
    return (oh1 * w1 + oh2 * w2) / (w1 + w2 + jnp.float32(1e-20))


def _moe_kernel(xb_ref, gate_w_ref, e_bias_ref, wg_ref, wu_ref, wd_ref,
                sg_ref, su_ref, sd_ref, out_ref, we_ref):
    s = pl.program_id(0)

    @pl.when(s == 0)
    def _route():
        we_ref[...] = _routing_weights(
            xb_ref[...], gate_w_ref[...], e_bias_ref[...], T)

    @pl.when(s > 0)
    def _expert_and_shared_slice():
        e = s - 1
        wg = wg_ref[0].astype(jnp.bfloat16)
        wu = wu_ref[0].astype(jnp.bfloat16)
        wd = wd_ref[0].astype(jnp.bfloat16)
        sg = sg_ref[...].astype(jnp.bfloat16)
        su = su_ref[...].astype(jnp.bfloat16)
        sd = sd_ref[...].astype(jnp.bfloat16)
        col = jax.lax.broadcasted_iota(jnp.int32, (Q, E), 1)
        for j in range(NQ):
            row = j * Q
            xb = xb_ref[pl.ds(row, Q), :]
            g = _dot_t(xb, wg)
            u = _dot_t(xb, wu)
            h = (jax.nn.silu(g) * u).astype(jnp.bfloat16)
            d = _dot_t(h, wd)
            wsel = jnp.sum(
                jnp.where(col == e, we_ref[pl.ds(row, Q), :], jnp.float32(0.0)),
                axis=1, keepdims=True)
            contrib = jnp.float32(ROUTED_SCALING) * wsel * d
            gsh = _dot_t(xb, sg)
            ush = _dot_t(xb, su)
            hsh = (jax.nn.silu(gsh) * ush).astype(jnp.bfloat16)
            contrib = contrib + _dot_t(hsh, sd)

            @pl.when(s == 1)
            def _init():
                out_ref[pl.ds(row, Q), :] = contrib

            @pl.when(s > 1)
            def _acc():
                out_ref[pl.ds(row, Q), :] += contrib


def kernel(hidden_states, gate_w, e_bias, w_gate, w_up, w_down,
           sw_gate, sw_up, sw_down):
    xb = hidden_states.astype(jnp.bfloat16)
    e_bias2 = e_bias.reshape(1, E)
    const2 = lambda s: (0, 0)
    expert_ix = lambda s: (jnp.maximum(s - 1, 0), 0, 0)
    slice_ix = lambda s: (jnp.maximum(s - 1, 0), 0)
    out = pl.pallas_call(
        _moe_kernel,
        grid=(1 + E,),
        in_specs=[
            pl.BlockSpec((T, H), const2),
            pl.BlockSpec((E, H), const2),
            pl.BlockSpec((1, E), const2),
            pl.BlockSpec((1, I, H), expert_ix),
            pl.BlockSpec((1, I, H), expert_ix),
            pl.BlockSpec((1, H, I), expert_ix),
            pl.BlockSpec((SCH, H), slice_ix),
            pl.BlockSpec((SCH, H), slice_ix),
            pl.BlockSpec((H, SCH), lambda s: (0, jnp.maximum(s - 1, 0))),
        ],
        out_specs=pl.BlockSpec((T, H), const2),
        out_shape=jax.ShapeDtypeStruct((T, H), jnp.float32),
        scratch_shapes=[
            pltpu.VMEM((T, E), jnp.float32),
        ],
        compiler_params=pltpu.CompilerParams(
            dimension_semantics=("arbitrary",),
        ),
    )(xb, gate_w, e_bias2, w_gate, w_up, w_down,
      sw_gate, sw_up, sw_down)
    return out


# speedup vs baseline: 1.1248x; 1.1248x over previous
import jax
import jax.numpy as jnp
from jax.experimental import pallas as pl
from jax.experimental.pallas import tpu as pltpu

T = 2048
H = 1024
E = 8
I = 512
ISH = 1024
ROUTED_SCALING = 2.5

NQ = 2
Q = T // NQ


def _dot_t(a, b):
    return jax.lax.dot_general(
        a, b, (((1,), (1,)), ((), ())), preferred_element_type=jnp.float32)


def _argmax8_lowest(vals):
    best = vals[:, 0:1]
    bidx = jnp.zeros_like(best, dtype=jnp.int32)
    for c in range(1, E):
        v = vals[:, c : c + 1]
        take = v > best
        bidx = jnp.where(take, jnp.int32(c), bidx)
        best = jnp.where(take, v, best)
    return bidx


def _routing_weights(xb, gate_w, e_bias):
    logits = _dot_t(xb, gate_w.astype(jnp.bfloat16))
    scores = jax.nn.sigmoid(logits)
    s4c = scores + e_bias

    def top2sum(g):
        cols = [g[:, c : c + 1] for c in range(4)]
        ps = [cols[i] + cols[j] for i in range(4) for j in range(i + 1, 4)]
        out = ps[0]
        for p in ps[1:]:
            out = jnp.maximum(out, p)
        return out

    one = jnp.float32(1.0)
    zero = jnp.float32(0.0)
    use_g0 = jnp.where(top2sum(s4c[:, 0:4]) >= top2sum(s4c[:, 4:8]), one, zero)
    col = jax.lax.broadcasted_iota(jnp.int32, (Q, E), 1)
    in_g0 = jnp.where(col < 4, one, zero)
    group_mask = in_g0 * use_g0 + (one - in_g0) * (one - use_g0)
    neg_inf = jnp.float32(-jnp.inf)
    masked = jnp.where(group_mask > 0.5, s4c, neg_inf)

    idx1 = _argmax8_lowest(masked)
    idx2 = _argmax8_lowest(jnp.where(col == idx1, neg_inf, masked))
    oh1 = jnp.where(col == idx1, one, zero)
    oh2 = jnp.where(col == idx2, one, zero)
    w1 = jnp.sum(oh1 * scores, axis=1, keepdims=True)
    w2 = jnp.sum(oh2 * scores, axis=1, keepdims=True)
    return (oh1 * w1 + oh2 * w2) / (w1 + w2 + jnp.float32(1e-20))


def _moe_kernel(xb_ref, gate_w_ref, e_bias_ref, wg_ref, wu_ref, wd_ref,
                sg_any, su_any, sd_any, out_ref,
                we_ref, sg_vm, su_vm, sd_vm, sem_g, sem_u, sem_d):
    s = pl.program_id(0)

    @pl.when(s == 0)
    def _route_and_start_copies():
        pltpu.make_async_copy(sg_any, sg_vm, sem_g).start()
        pltpu.make_async_copy(su_any, su_vm, sem_u).start()
        pltpu.make_async_copy(sd_any, sd_vm, sem_d).start()
        for j in range(NQ):
            row = j * Q
            we_ref[pl.ds(row, Q), :] = _routing_weights(
                xb_ref[pl.ds(row, Q), :], gate_w_ref[...], e_bias_ref[...])

    @pl.when((s > 0) & (s <= E))
    def _expert():
        e = s - 1
        wg = wg_ref[0].astype(jnp.bfloat16)
        wu = wu_ref[0].astype(jnp.bfloat16)
        wd = wd_ref[0].astype(jnp.bfloat16)
        col = jax.lax.broadcasted_iota(jnp.int32, (Q, E), 1)
        for j in range(NQ):
            row = j * Q
            xb = xb_ref[pl.ds(row, Q), :]
            g = _dot_t(xb, wg)
            u = _dot_t(xb, wu)
            h = (jax.nn.silu(g) * u).astype(jnp.bfloat16)
            d = _dot_t(h, wd)
            wsel = jnp.sum(
                jnp.where(col == e, we_ref[pl.ds(row, Q), :], jnp.float32(0.0)),
                axis=1, keepdims=True)
            contrib = jnp.float32(ROUTED_SCALING) * wsel * d

            @pl.when(s == 1)
            def _init():
                out_ref[pl.ds(row, Q), :] = contrib

            @pl.when(s > 1)
            def _acc():
                out_ref[pl.ds(row, Q), :] += contrib

    @pl.when(s == E + 1)
    def _shared():
        pltpu.make_async_copy(sg_any, sg_vm, sem_g).wait()
        pltpu.make_async_copy(su_any, su_vm, sem_u).wait()
        pltpu.make_async_copy(sd_any, sd_vm, sem_d).wait()
        sgw = sg_vm[...].astype(jnp.bfloat16)
        suw = su_vm[...].astype(jnp.bfloat16)
        sdw = sd_vm[...].astype(jnp.bfloat16)
        for j in range(NQ):
            row = j * Q
            xb = xb_ref[pl.ds(row, Q), :]
            sg = _dot_t(xb, sgw)
            su = _dot_t(xb, suw)
            sh = (jax.nn.silu(sg) * su).astype(jnp.bfloat16)
            out_ref[pl.ds(row, Q), :] += _dot_t(sh, sdw)


def kernel(hidden_states, gate_w, e_bias, w_gate, w_up, w_down,
           sw_gate, sw_up, sw_down):
    xb = hidden_states.astype(jnp.bfloat16)
    e_bias2 = e_bias.reshape(1, E)
    const2 = lambda s: (0, 0)
    expert_ix = lambda s: (jnp.minimum(jnp.maximum(s - 1, 0), E - 1), 0, 0)
    out = pl.pallas_call(
        _moe_kernel,
        grid=(2 + E,),
        in_specs=[
            pl.BlockSpec((T, H), const2),
            pl.BlockSpec((E, H), const2),
            pl.BlockSpec((1, E), const2),
            pl.BlockSpec((1, I, H), expert_ix),
            pl.BlockSpec((1, I, H), expert_ix),
            pl.BlockSpec((1, H, I), expert_ix),
            pl.BlockSpec(memory_space=pl.ANY),
            pl.BlockSpec(memory_space=pl.ANY),
            pl.BlockSpec(memory_space=pl.ANY),
        ],
        out_specs=pl.BlockSpec((T, H), const2),
        out_shape=jax.ShapeDtypeStruct((T, H), jnp.float32),
        scratch_shapes=[
            pltpu.VMEM((T, E), jnp.float32),
            pltpu.VMEM((ISH, H), jnp.float32),
            pltpu.VMEM((ISH, H), jnp.float32),
            pltpu.VMEM((H, ISH), jnp.float32),
            pltpu.SemaphoreType.DMA,
            pltpu.SemaphoreType.DMA,
            pltpu.SemaphoreType.DMA,
        ],
        compiler_params=pltpu.CompilerParams(
            dimension_semantics=("arbitrary",),
        ),
    )(xb, gate_w, e_bias2, w_gate, w_up, w_down, sw_gate, sw_up, sw_down)
    return out


# speedup vs baseline: 1.3070x; 1.1619x over previous
import jax
import jax.numpy as jnp
from jax.experimental import pallas as pl
from jax.experimental.pallas import tpu as pltpu

T = 2048
H = 1024
E = 8
I = 512
ISH = 1024
ROUTED_SCALING = 2.5

NQ = 2
Q = T // NQ


def _dot_t(a, b):
    return jax.lax.dot_general(
        a, b, (((1,), (1,)), ((), ())), preferred_element_type=jnp.float32)


def _argmax8_lowest(vals):
    best = vals[:, 0:1]
    bidx = jnp.zeros_like(best, dtype=jnp.int32)
    for c in range(1, E):
        v = vals[:, c : c + 1]
        take = v > best
        bidx = jnp.where(take, jnp.int32(c), bidx)
        best = jnp.where(take, v, best)
    return bidx


def _routing_weights(xb, gate_w, e_bias):
    logits = _dot_t(xb, gate_w.astype(jnp.bfloat16))
    scores = jax.nn.sigmoid(logits)
    s4c = scores + e_bias

    def top2sum(g):
        cols = [g[:, c : c + 1] for c in range(4)]
        ps = [cols[i] + cols[j] for i in range(4) for j in range(i + 1, 4)]
        out = ps[0]
        for p in ps[1:]:
            out = jnp.maximum(out, p)
        return out

    one = jnp.float32(1.0)
    zero = jnp.float32(0.0)
    use_g0 = jnp.where(top2sum(s4c[:, 0:4]) >= top2sum(s4c[:, 4:8]), one, zero)
    col = jax.lax.broadcasted_iota(jnp.int32, (Q, E), 1)
    in_g0 = jnp.where(col < 4, one, zero)
    group_mask = in_g0 * use_g0 + (one - in_g0) * (one - use_g0)
    neg_inf = jnp.float32(-jnp.inf)
    masked = jnp.where(group_mask > 0.5, s4c, neg_inf)

    idx1 = _argmax8_lowest(masked)
    idx2 = _argmax8_lowest(jnp.where(col == idx1, neg_inf, masked))
    oh1 = jnp.where(col == idx1, one, zero)
    oh2 = jnp.where(col == idx2, one, zero)
    w1 = jnp.sum(oh1 * scores, axis=1, keepdims=True)
    w2 = jnp.sum(oh2 * scores, axis=1, keepdims=True)
    return (oh1 * w1 + oh2 * w2) / (w1 + w2 + jnp.float32(1e-20))


def _moe_kernel(xb_ref, gate_w_ref, e_bias_ref, wg_ref, wu_ref, wd_ref,
                sg_ref, su_ref, sd_ref, out_ref, we_ref):
    s = pl.program_id(0)

    @pl.when(s == 0)
    def _prologue():
        for j in range(NQ):
            row = j * Q
            xb = xb_ref[pl.ds(row, Q), :]
            we_ref[pl.ds(row, Q), :] = _routing_weights(
                xb, gate_w_ref[...], e_bias_ref[...])
            sg = _dot_t(xb, sg_ref[...])
            su = _dot_t(xb, su_ref[...])
            sh = (jax.nn.silu(sg) * su).astype(jnp.bfloat16)
            out_ref[pl.ds(row, Q), :] = _dot_t(sh, sd_ref[...])

    @pl.when(s > 0)
    def _expert():
        e = s - 1
        wg = wg_ref[0].astype(jnp.bfloat16)
        wu = wu_ref[0].astype(jnp.bfloat16)
        wd = wd_ref[0].astype(jnp.bfloat16)
        col = jax.lax.broadcasted_iota(jnp.int32, (Q, E), 1)
        for j in range(NQ):
            row = j * Q
            xb = xb_ref[pl.ds(row, Q), :]
            g = _dot_t(xb, wg)
            u = _dot_t(xb, wu)
            h = (jax.nn.silu(g) * u).astype(jnp.bfloat16)
            d = _dot_t(h, wd)
            wsel = jnp.sum(
                jnp.where(col == e, we_ref[pl.ds(row, Q), :], jnp.float32(0.0)),
                axis=1, keepdims=True)
            out_ref[pl.ds(row, Q), :] += jnp.float32(ROUTED_SCALING) * wsel * d


def kernel(hidden_states, gate_w, e_bias, w_gate, w_up, w_down,
           sw_gate, sw_up, sw_down):
    xb = hidden_states.astype(jnp.bfloat16)
    e_bias2 = e_bias.reshape(1, E)
    const2 = lambda s: (0, 0)
    expert_ix = lambda s: (jnp.maximum(s - 1, 0), 0, 0)
    out = pl.pallas_call(
        _moe_kernel,
        grid=(1 + E,),
        in_specs=[
            pl.BlockSpec((T, H), const2),
            pl.BlockSpec((E, H), const2),
            pl.BlockSpec((1, E), const2),
            pl.BlockSpec((1, I, H), expert_ix),
            pl.BlockSpec((1, I, H), expert_ix),
            pl.BlockSpec((1, H, I), expert_ix),
            pl.BlockSpec((ISH, H), const2),
            pl.BlockSpec((ISH, H), const2),
            pl.BlockSpec((H, ISH), const2),
        ],
        out_specs=pl.BlockSpec((T, H), const2),
        out_shape=jax.ShapeDtypeStruct((T, H), jnp.float32),
        scratch_shapes=[
            pltpu.VMEM((T, E), jnp.float32),
        ],
        compiler_params=pltpu.CompilerParams(
            dimension_semantics=("arbitrary",),
        ),
    )(xb, gate_w, e_bias2, w_gate, w_up, w_down, sw_gate, sw_up, sw_down)
    return out
